# baseline (speedup 1.0000x reference)
import functools
import jax
import jax.numpy as jnp
from jax import lax
from jax.experimental import pallas as pl
from jax.experimental.pallas import tpu as pltpu
from jax.experimental.pallas import tpu_sc as plsc

B = 16
C = 4096


WIN = 2048
NB = WIN // 128


def _sc_body(n, ep, ne16, ne32, x2_hbm, src_hbm, dst_hbm, agg_hbm, deg_hbm,
             ws0, wd0, ws1, wd1, e0, stage, stage2, acc_sh, *bufs):
    rowsb = list(bufs[0:8])
    semg = list(bufs[8:16])
    sems = list(bufs[16:24])
    semd = bufs[24]
    seme0 = bufs[25]
    seme1 = bufs[26]
    semh = (bufs[27], bufs[28])
    core = lax.axis_index("c")
    sub = lax.axis_index("s")
    rpt = n // 16
    base = sub * rpt

    lane = lax.iota(jnp.int32, 16)
    one0 = jnp.where(lane == 0, jnp.full((16,), 1.0, jnp.float32),
                     jnp.zeros((16,), jnp.float32))
    zv = jnp.zeros((16,), jnp.float32)

    def init_row(i, _):
        e0[i, pl.ds(0, 16)] = one0
        return 0

    lax.fori_loop(0, 128, init_row, 0)

    def zero_stage(i, _):
        stage[i, pl.ds(0, 16)] = zv
        return 0

    def zero_acc():
        lax.fori_loop(0, 125, zero_stage, 0)
        for j in range(50):
            pltpu.async_copy(stage, acc_sh.at[pl.ds(base + j * 125, 125), :],
                             semd)
        for j in range(50):
            pltpu.make_async_copy(x2_hbm.at[pl.ds(0, 125)], stage, semd).wait()

    def _wait(sem, buf):
        pltpu.make_async_copy(x2_hbm.at[pl.ds(0, 128)], buf, sem).wait()

    def agg_window(wsrc, wdst):
        for r in range(4):
            pltpu.async_copy(x2_hbm.at[wsrc.at[pl.ds(r * 128, 128)]],
                             rowsb[r], semg[r])
        for r in range(NB):
            b = r % 8
            if r + 4 < NB:
                nb_ = (r + 4) % 8
                if r + 4 >= 8:
                    _wait(sems[nb_], rowsb[nb_])
                pltpu.async_copy(
                    x2_hbm.at[wsrc.at[pl.ds((r + 4) * 128, 128)]],
                    rowsb[nb_], semg[nb_])
            _wait(semg[b], rowsb[b])
            pltpu.async_copy(rowsb[b], acc_sh.at[wdst.at[r]], sems[b],
                             add=True)
        for r in range(NB - 8, NB):
            b = r % 8
            _wait(sems[b], rowsb[b])

    def deg_window(wdst):
        for r in range(NB):
            pltpu.async_copy(e0, acc_sh.at[wdst.at[r]], semd, add=True)
        for r in range(NB):
            _wait(semd, e0)

    def drain(out_hbm, dst0):
        stg = (stage, stage2)
        for j in range(50):
            b = j % 2
            if j >= 2:
                pltpu.make_async_copy(x2_hbm.at[pl.ds(0, 125)], stg[b],
                                      semh[b]).wait()
            pltpu.sync_copy(acc_sh.at[pl.ds(base + j * 125, 125), :], stg[b])
            pltpu.async_copy(stg[b],
                             out_hbm.at[pl.ds(dst0 + base + j * 125, 125), :],
                             semh[b])
        for b in range(2):
            pltpu.make_async_copy(x2_hbm.at[pl.ds(0, 125)], stg[b],
                                  semh[b]).wait()

    def edge_pass(sbase, ebase, nwin, is_deg):
        def fetch(w, wsrc, wdst, sem):
            if not is_deg:
                pltpu.async_copy(
                    src_hbm.at[pl.ds(sbase + ebase + w * WIN, WIN)], wsrc, sem)
            pltpu.async_copy(
                dst_hbm.at[pl.ds((ebase + w * WIN) // 128, NB), :], wdst, sem)

        def dr(wsrc, wdst, sem):
            if not is_deg:
                pltpu.make_async_copy(src_hbm.at[pl.ds(0, WIN)], wsrc,
                                      sem).wait()
            pltpu.make_async_copy(dst_hbm.at[pl.ds(0, NB), :], wdst,
                                  sem).wait()

        fetch(0, ws0, wd0, seme0)

        def win_body(w, _):
            @pl.when(jnp.logical_and(w + 1 < nwin, (w + 1) % 2 == 0))
            def _():
                fetch(w + 1, ws0, wd0, seme0)

            @pl.when(jnp.logical_and(w + 1 < nwin, (w + 1) % 2 == 1))
            def _():
                fetch(w + 1, ws1, wd1, seme1)

            @pl.when(w % 2 == 0)
            def _():
                dr(ws0, wd0, seme0)
                if is_deg:
                    deg_window(wd0)
                else:
                    agg_window(ws0, wd0)

            @pl.when(w % 2 == 1)
            def _():
                dr(ws1, wd1, seme1)
                if is_deg:
                    deg_window(wd1)
                else:
                    agg_window(ws1, wd1)

            return 0

        lax.fori_loop(0, nwin, win_body, 0)

    for p in range(4):
        s = core * 4 + p
        zero_acc()
        plsc.subcore_barrier()
        edge_pass(s * ep, sub * ne16, ne16 // WIN, False)
        plsc.subcore_barrier()
        drain(agg_hbm, s * n)
        plsc.subcore_barrier()

    zero_acc()
    plsc.subcore_barrier()
    edge_pass(0, (core * 16 + sub) * ne32, ne32 // WIN, True)
    plsc.subcore_barrier()
    drain(deg_hbm, core * n)


def _sc_aggregate(x, edge_index):
    n, d = x.shape
    e = edge_index.shape[1]
    ne16 = -(-(-(-e // 16)) // WIN) * WIN
    ne32 = -(-(-(-e // 32)) // WIN) * WIN
    ep = max(16 * ne16, 32 * ne32)
    npad = ep - e

    ar = jnp.arange(npad, dtype=jnp.int32)
    srcp = jnp.concatenate([edge_index[0], (ar * 37) % n])
    dstp = jnp.concatenate([edge_index[1], n + (ar % 16)])
    src8 = (srcp[None, :]
            + (jnp.arange(8, dtype=jnp.int32) * n)[:, None]).reshape(-1)
    dst2 = dstp.reshape(ep // 128, 128)
    x2 = x.reshape(n, 8, 16).transpose(1, 0, 2).reshape(8 * n, 16)

    fn = functools.partial(_sc_body, n, ep, ne16, ne32)
    agg, deg = pl.kernel(
        fn,
        mesh=plsc.VectorSubcoreMesh(core_axis_name="c", subcore_axis_name="s"),
        compiler_params=pltpu.CompilerParams(use_tc_tiling_on_sc=False),
        out_type=[
            jax.ShapeDtypeStruct((8 * n, 16), jnp.float32),
            jax.ShapeDtypeStruct((2 * n, 16), jnp.float32),
        ],
        scratch_types=[
            pltpu.VMEM((WIN,), jnp.int32),
            pltpu.VMEM((NB, 128), jnp.int32),
            pltpu.VMEM((WIN,), jnp.int32),
            pltpu.VMEM((NB, 128), jnp.int32),
            pltpu.VMEM((128, 16), jnp.float32),
            pltpu.VMEM((125, 16), jnp.float32),
            pltpu.VMEM((125, 16), jnp.float32),
            pltpu.VMEM_SHARED((n + 16, 16), jnp.float32),
        ] + [pltpu.VMEM((128, 16), jnp.float32) for _ in range(8)]
        + [pltpu.SemaphoreType.DMA for _ in range(27)],
    )(x2, src8, dst2)
    return agg, deg


def _pick_rows(n):
    for r in (2500, 2000, 1000, 500, 250, 125, 25, 8, 5, 1):
        if n % r == 0 and (r % 8 == 0 or r == n):
            return r
    return 1


def _fuse_body(*refs):
    (x_ref, a_ref, d_ref, bid_ref, w_ref, b_ref,
     h_ref, sums_ref, cnts_ref) = refs
    i = pl.program_id(0)
    x = x_ref[...]
    w = w_ref[...]
    acc = jnp.dot(x, w, preferred_element_type=jnp.float32)
    deg = d_ref[0][:, 0:1] + d_ref[1][:, 0:1]
    aw = None
    for s in range(8):
        t = jnp.dot(a_ref[s], w[16 * s:16 * (s + 1), :],
                    preferred_element_type=jnp.float32)
        aw = t if aw is None else aw + t
    h = acc + aw / jnp.maximum(deg, 1.0) + b_ref[...]
    h = jnp.maximum(h, 0.0)
    h_ref[...] = h
    r = x.shape[0]
    onehot = (bid_ref[...] == lax.broadcasted_iota(jnp.int32, (r, B), 1))
    onehot = onehot.astype(jnp.float32)
    part = lax.dot_general(onehot, h, (((0,), (0,)), ((), ())),
                           preferred_element_type=jnp.float32)
    cnt = jnp.sum(onehot, axis=0)[:, None]

    @pl.when(i == 0)
    def _():
        sums_ref[...] = jnp.zeros_like(sums_ref)
        cnts_ref[...] = jnp.zeros_like(cnts_ref)

    sums_ref[...] += part
    cnts_ref[...] += jnp.broadcast_to(cnt, cnts_ref.shape)


def _bcast_body(h_ref, bid_ref, sums_ref, cnts_ref, out_ref):
    gmean = sums_ref[...] / jnp.maximum(cnts_ref[...], 1.0)
    r = h_ref.shape[0]
    onehot = (bid_ref[...] == lax.broadcasted_iota(jnp.int32, (r, B), 1))
    onehot = onehot.astype(jnp.float32)
    out_ref[...] = h_ref[...] + jnp.dot(onehot, gmean,
                                        preferred_element_type=jnp.float32)


def _dense_stages(x, agg, deg2, batch_ids, W, b):
    n, d = x.shape
    r = _pick_rows(n)
    nb = n // r
    bid2 = batch_ids.reshape(n, 1)
    b2 = b.reshape(1, d)

    agg3 = agg.reshape(8, n, 16)
    deg3 = deg2.reshape(2, n, 16)

    h, sums, cnts = pl.pallas_call(
        _fuse_body,
        grid=(nb,),
        in_specs=[
            pl.BlockSpec((r, d), lambda i: (i, 0)),
            pl.BlockSpec((8, r, 16), lambda i: (0, i, 0)),
            pl.BlockSpec((2, r, 16), lambda i: (0, i, 0)),
            pl.BlockSpec((r, 1), lambda i: (i, 0)),
            pl.BlockSpec((d, d), lambda i: (0, 0)),
            pl.BlockSpec((1, d), lambda i: (0, 0)),
        ],
        out_specs=[
            pl.BlockSpec((r, d), lambda i: (i, 0)),
            pl.BlockSpec((B, d), lambda i: (0, 0)),
            pl.BlockSpec((B, d), lambda i: (0, 0)),
        ],
        out_shape=[
            jax.ShapeDtypeStruct((n, d), jnp.float32),
            jax.ShapeDtypeStruct((B, d), jnp.float32),
            jax.ShapeDtypeStruct((B, d), jnp.float32),
        ],
    )(x, agg3, deg3, bid2, W, b2)

    out = pl.pallas_call(
        _bcast_body,
        grid=(nb,),
        in_specs=[
            pl.BlockSpec((r, d), lambda i: (i, 0)),
            pl.BlockSpec((r, 1), lambda i: (i, 0)),
            pl.BlockSpec((B, d), lambda i: (0, 0)),
            pl.BlockSpec((B, d), lambda i: (0, 0)),
        ],
        out_specs=pl.BlockSpec((r, d), lambda i: (i, 0)),
        out_shape=jax.ShapeDtypeStruct((n, d), jnp.float32),
    )(h, bid2, sums, cnts)
    return out


def kernel(x, edge_index, batch_ids, W, b):
    agg, deg2 = _sc_aggregate(x, edge_index)
    return _dense_stages(x, agg, deg2, batch_ids, W, b)


# speedup vs baseline: 1.4145x; 1.4145x over previous
import functools
import jax
import jax.numpy as jnp
from jax import lax
from jax.experimental import pallas as pl
from jax.experimental.pallas import tpu as pltpu
from jax.experimental.pallas import tpu_sc as plsc

B = 16
C = 4096


WIN = 2048
NB = WIN // 128


def _sc_body(n, ep, ne16, ne32, x2_hbm, src_hbm, dst_hbm, agg_hbm, deg_hbm,
             ws0, wd0, ws1, wd1, e0, stage, stage2, acc_sh, *bufs):
    rowsb = list(bufs[0:8])
    semg = list(bufs[8:16])
    sems = list(bufs[16:24])
    semd = bufs[24]
    seme0 = bufs[25]
    seme1 = bufs[26]
    semh = (bufs[27], bufs[28])
    core = lax.axis_index("c")
    sub = lax.axis_index("s")
    rpt = n // 16
    base = sub * rpt

    lane = lax.iota(jnp.int32, 16)
    one0 = jnp.where(lane == 0, jnp.full((16,), 1.0, jnp.float32),
                     jnp.zeros((16,), jnp.float32))
    zv = jnp.zeros((16,), jnp.float32)

    def init_row(i, _):
        e0[i, pl.ds(0, 16)] = one0
        return 0

    lax.fori_loop(0, 128, init_row, 0)

    def zero_stage(i, _):
        stage[i, pl.ds(0, 16)] = zv
        return 0

    def zero_acc():
        lax.fori_loop(0, 125, zero_stage, 0)
        for j in range(50):
            pltpu.async_copy(stage, acc_sh.at[pl.ds(base + j * 125, 125), :],
                             semd)
        for j in range(50):
            pltpu.make_async_copy(x2_hbm.at[pl.ds(0, 125)], stage, semd).wait()

    def _wait(sem, buf):
        pltpu.make_async_copy(x2_hbm.at[pl.ds(0, 128)], buf, sem).wait()

    def agg_window(wsrc, wdst):
        for r in range(4):
            pltpu.async_copy(x2_hbm.at[wsrc.at[pl.ds(r * 128, 128)]],
                             rowsb[r], semg[r])
        for r in range(NB):
            b = r % 8
            if r + 4 < NB:
                nb_ = (r + 4) % 8
                if r + 4 >= 8:
                    _wait(sems[nb_], rowsb[nb_])
                pltpu.async_copy(
                    x2_hbm.at[wsrc.at[pl.ds((r + 4) * 128, 128)]],
                    rowsb[nb_], semg[nb_])
            _wait(semg[b], rowsb[b])
            pltpu.async_copy(rowsb[b], acc_sh.at[wdst.at[r]], sems[b],
                             add=True)
        for r in range(NB - 8, NB):
            b = r % 8
            _wait(sems[b], rowsb[b])

    def deg_window(wdst):
        for r in range(NB):
            pltpu.async_copy(e0, acc_sh.at[wdst.at[r]], semd, add=True)
        for r in range(NB):
            _wait(semd, e0)

    def drain(out_hbm, dst0):
        stg = (stage, stage2)
        for j in range(50):
            b = j % 2
            if j >= 2:
                pltpu.make_async_copy(x2_hbm.at[pl.ds(0, 125)], stg[b],
                                      semh[b]).wait()
            pltpu.sync_copy(acc_sh.at[pl.ds(base + j * 125, 125), :], stg[b])
            pltpu.async_copy(stg[b],
                             out_hbm.at[pl.ds(dst0 + base + j * 125, 125), :],
                             semh[b])
        for b in range(2):
            pltpu.make_async_copy(x2_hbm.at[pl.ds(0, 125)], stg[b],
                                  semh[b]).wait()

    def edge_pass(sbase, ebase, nwin, is_deg):
        def fetch(w, wsrc, wdst, sem):
            if not is_deg:
                pltpu.async_copy(
                    src_hbm.at[pl.ds(sbase + ebase + w * WIN, WIN)], wsrc, sem)
            pltpu.async_copy(
                dst_hbm.at[pl.ds((ebase + w * WIN) // 128, NB), :], wdst, sem)

        def dr(wsrc, wdst, sem):
            if not is_deg:
                pltpu.make_async_copy(src_hbm.at[pl.ds(0, WIN)], wsrc,
                                      sem).wait()
            pltpu.make_async_copy(dst_hbm.at[pl.ds(0, NB), :], wdst,
                                  sem).wait()

        fetch(0, ws0, wd0, seme0)

        def win_body(w, _):
            @pl.when(jnp.logical_and(w + 1 < nwin, (w + 1) % 2 == 0))
            def _():
                fetch(w + 1, ws0, wd0, seme0)

            @pl.when(jnp.logical_and(w + 1 < nwin, (w + 1) % 2 == 1))
            def _():
                fetch(w + 1, ws1, wd1, seme1)

            @pl.when(w % 2 == 0)
            def _():
                dr(ws0, wd0, seme0)
                if is_deg:
                    deg_window(wd0)
                else:
                    agg_window(ws0, wd0)

            @pl.when(w % 2 == 1)
            def _():
                dr(ws1, wd1, seme1)
                if is_deg:
                    deg_window(wd1)
                else:
                    agg_window(ws1, wd1)

            return 0

        lax.fori_loop(0, nwin, win_body, 0)

    for p in range(4):
        s = core * 4 + p
        zero_acc()
        plsc.subcore_barrier()
        edge_pass(s * ep, sub * ne16, ne16 // WIN, False)
        plsc.subcore_barrier()
        drain(agg_hbm, s * n)
        plsc.subcore_barrier()

    zero_acc()
    plsc.subcore_barrier()
    edge_pass(0, (core * 16 + sub) * ne32, ne32 // WIN, True)
    plsc.subcore_barrier()
    drain(deg_hbm, core * n)


def _sc_aggregate(x, edge_index):
    n, d = x.shape
    e = edge_index.shape[1]
    ne16 = -(-(-(-e // 16)) // WIN) * WIN
    ne32 = -(-(-(-e // 32)) // WIN) * WIN
    ep = max(16 * ne16, 32 * ne32)
    npad = ep - e

    ar = jnp.arange(npad, dtype=jnp.int32)
    srcp = jnp.concatenate([edge_index[0], (ar * 37) % n])
    dstp = jnp.concatenate([edge_index[1], n + (ar % 16)])
    src8 = (srcp[None, :] * 8
            + jnp.arange(8, dtype=jnp.int32)[:, None]).reshape(-1)
    dst2 = dstp.reshape(ep // 128, 128)
    x2 = x.reshape(8 * n, 16)

    fn = functools.partial(_sc_body, n, ep, ne16, ne32)
    agg, deg = pl.kernel(
        fn,
        mesh=plsc.VectorSubcoreMesh(core_axis_name="c", subcore_axis_name="s"),
        compiler_params=pltpu.CompilerParams(use_tc_tiling_on_sc=False),
        out_type=[
            jax.ShapeDtypeStruct((8 * n, 16), jnp.float32),
            jax.ShapeDtypeStruct((2 * n, 16), jnp.float32),
        ],
        scratch_types=[
            pltpu.VMEM((WIN,), jnp.int32),
            pltpu.VMEM((NB, 128), jnp.int32),
            pltpu.VMEM((WIN,), jnp.int32),
            pltpu.VMEM((NB, 128), jnp.int32),
            pltpu.VMEM((128, 16), jnp.float32),
            pltpu.VMEM((125, 16), jnp.float32),
            pltpu.VMEM((125, 16), jnp.float32),
            pltpu.VMEM_SHARED((n + 16, 16), jnp.float32),
        ] + [pltpu.VMEM((128, 16), jnp.float32) for _ in range(8)]
        + [pltpu.SemaphoreType.DMA for _ in range(27)],
    )(x2, src8, dst2)
    return agg, deg


def _pick_rows(n):
    for r in (2500, 2000, 1000, 500, 250, 125, 25, 8, 5, 1):
        if n % r == 0 and (r % 8 == 0 or r == n):
            return r
    return 1


def _fuse_body(*refs):
    (x_ref, a_ref, d_ref, bid_ref, w_ref, b_ref,
     h_ref, sums_ref, cnts_ref) = refs
    i = pl.program_id(0)
    x = x_ref[...]
    w = w_ref[...]
    acc = jnp.dot(x, w, preferred_element_type=jnp.float32)
    deg = d_ref[0][:, 0:1] + d_ref[1][:, 0:1]
    aw = None
    for s in range(8):
        t = jnp.dot(a_ref[s], w[16 * s:16 * (s + 1), :],
                    preferred_element_type=jnp.float32)
        aw = t if aw is None else aw + t
    h = acc + aw / jnp.maximum(deg, 1.0) + b_ref[...]
    h = jnp.maximum(h, 0.0)
    h_ref[...] = h
    r = x.shape[0]
    onehot = (bid_ref[...] == lax.broadcasted_iota(jnp.int32, (r, B), 1))
    onehot = onehot.astype(jnp.float32)
    part = lax.dot_general(onehot, h, (((0,), (0,)), ((), ())),
                           preferred_element_type=jnp.float32)
    cnt = jnp.sum(onehot, axis=0)[:, None]

    @pl.when(i == 0)
    def _():
        sums_ref[...] = jnp.zeros_like(sums_ref)
        cnts_ref[...] = jnp.zeros_like(cnts_ref)

    sums_ref[...] += part
    cnts_ref[...] += jnp.broadcast_to(cnt, cnts_ref.shape)


def _bcast_body(h_ref, bid_ref, sums_ref, cnts_ref, out_ref):
    gmean = sums_ref[...] / jnp.maximum(cnts_ref[...], 1.0)
    r = h_ref.shape[0]
    onehot = (bid_ref[...] == lax.broadcasted_iota(jnp.int32, (r, B), 1))
    onehot = onehot.astype(jnp.float32)
    out_ref[...] = h_ref[...] + jnp.dot(onehot, gmean,
                                        preferred_element_type=jnp.float32)


def _dense_stages(x, agg, deg2, batch_ids, W, b):
    n, d = x.shape
    r = _pick_rows(n)
    nb = n // r
    bid2 = batch_ids.reshape(n, 1)
    b2 = b.reshape(1, d)

    agg3 = agg.reshape(8, n, 16)
    deg3 = deg2.reshape(2, n, 16)

    h, sums, cnts = pl.pallas_call(
        _fuse_body,
        grid=(nb,),
        in_specs=[
            pl.BlockSpec((r, d), lambda i: (i, 0)),
            pl.BlockSpec((8, r, 16), lambda i: (0, i, 0)),
            pl.BlockSpec((2, r, 16), lambda i: (0, i, 0)),
            pl.BlockSpec((r, 1), lambda i: (i, 0)),
            pl.BlockSpec((d, d), lambda i: (0, 0)),
            pl.BlockSpec((1, d), lambda i: (0, 0)),
        ],
        out_specs=[
            pl.BlockSpec((r, d), lambda i: (i, 0)),
            pl.BlockSpec((B, d), lambda i: (0, 0)),
            pl.BlockSpec((B, d), lambda i: (0, 0)),
        ],
        out_shape=[
            jax.ShapeDtypeStruct((n, d), jnp.float32),
            jax.ShapeDtypeStruct((B, d), jnp.float32),
            jax.ShapeDtypeStruct((B, d), jnp.float32),
        ],
    )(x, agg3, deg3, bid2, W, b2)

    out = pl.pallas_call(
        _bcast_body,
        grid=(nb,),
        in_specs=[
            pl.BlockSpec((r, d), lambda i: (i, 0)),
            pl.BlockSpec((r, 1), lambda i: (i, 0)),
            pl.BlockSpec((B, d), lambda i: (0, 0)),
            pl.BlockSpec((B, d), lambda i: (0, 0)),
        ],
        out_specs=pl.BlockSpec((r, d), lambda i: (i, 0)),
        out_shape=jax.ShapeDtypeStruct((n, d), jnp.float32),
    )(h, bid2, sums, cnts)
    return out


def kernel(x, edge_index, batch_ids, W, b):
    agg, deg2 = _sc_aggregate(x, edge_index)
    return _dense_stages(x, agg, deg2, batch_ids, W, b)
